# baseline (speedup 1.0000x reference)
import jax
import jax.numpy as jnp
from jax.experimental import pallas as pl
from jax.experimental.pallas import tpu as pltpu

_K = 10
_N = 2048
_R = 256
_NB = _N // _R
_L = 128
_INTERPRET = False


def _build_feats(pall, fhi_ref, flo_ref):
    x = pall[:, 0:1]
    y = pall[:, 1:2]
    z = pall[:, 2:3]
    f = jnp.concatenate(
        [x, y, z, x * x, y * y, z * z, x * y, x * z, y * z,
         jnp.ones((_N, 1), jnp.float32),
         jnp.zeros((_N, _L - 10), jnp.float32)], axis=1)
    hi = f.astype(jnp.bfloat16)
    fhi_ref[...] = hi
    flo_ref[...] = (f - hi.astype(jnp.float32)).astype(jnp.bfloat16)


def _normals(prow, pallT, fhi_ref, flo_ref):
    g = jax.lax.dot_general(prow, pallT, (((1,), (0,)), ((), ())),
                            preferred_element_type=jnp.float32)
    sq_row = jnp.sum(prow * prow, axis=1, keepdims=True)
    sq_all = jnp.sum(pallT * pallT, axis=0, keepdims=True)
    d = sq_row - 2.0 * g + sq_all

    dr = d.reshape(_R, _N // 128, 128)
    lo = jnp.minimum(dr[:, 0, :], dr[:, 1, :])
    hi = jnp.maximum(dr[:, 0, :], dr[:, 1, :])
    for j in range(2, _N // 128):
        v = dr[:, j, :]
        hi = jnp.minimum(hi, jnp.maximum(lo, v))
        lo = jnp.minimum(lo, v)
    w = jnp.concatenate([lo, hi], axis=1)
    t = jnp.min(w, axis=1, keepdims=True)
    for _ in range(_K - 1):
        t = jnp.min(jnp.where(w > t, w, jnp.inf), axis=1, keepdims=True)
    mask = (d <= t).astype(jnp.bfloat16)

    dims = (((1,), (0,)), ((), ()))
    s = (jax.lax.dot_general(mask, fhi_ref[...], dims,
                             preferred_element_type=jnp.float32)
         + jax.lax.dot_general(mask, flo_ref[...], dims,
                               preferred_element_type=jnp.float32))

    st = jnp.transpose(s)
    inv_c = 1.0 / st[9:10, :]
    mx = st[0:1, :] * inv_c
    my = st[1:2, :] * inv_c
    mz = st[2:3, :] * inv_c
    cxx = st[3:4, :] * inv_c - mx * mx
    cyy = st[4:5, :] * inv_c - my * my
    czz = st[5:6, :] * inv_c - mz * mz
    cxy = st[6:7, :] * inv_c - mx * my
    cxz = st[7:8, :] * inv_c - mx * mz
    cyz = st[8:9, :] * inv_c - my * mz

    q = (cxx + cyy + czz) * (1.0 / 3.0)
    aa = cxx - q
    bb = cyy - q
    cc = czz - q
    p2 = aa * aa + bb * bb + cc * cc + 2.0 * (cxy * cxy + cxz * cxz + cyz * cyz)
    p = jnp.sqrt(p2 * (1.0 / 6.0) + 1e-38)
    pinv = 1.0 / p
    b11 = aa * pinv
    b22 = bb * pinv
    b33 = cc * pinv
    b12 = cxy * pinv
    b13 = cxz * pinv
    b23 = cyz * pinv
    detb = (b11 * (b22 * b33 - b23 * b23)
            - b12 * (b12 * b33 - b23 * b13)
            + b13 * (b12 * b23 - b22 * b13))
    r = jnp.clip(detb * 0.5, -1.0, 1.0)
    lam = jnp.full_like(r, -2.0)
    for _ in range(12):
        f = lam * lam * lam - 3.0 * lam - 2.0 * r
        fp = 3.0 * lam * lam - 3.0 + 1e-10
        lam = lam - f / fp
    lmin = q + p * lam

    m11 = cxx - lmin
    m22 = cyy - lmin
    m33 = czz - lmin
    c12x = cxy * cyz - cxz * m22
    c12y = cxz * cxy - m11 * cyz
    c12z = m11 * m22 - cxy * cxy
    c13x = cxy * m33 - cxz * cyz
    c13y = cxz * cxz - m11 * m33
    c13z = m11 * cyz - cxy * cxz
    c23x = m22 * m33 - cyz * cyz
    c23y = cyz * cxz - cxy * m33
    c23z = cxy * cyz - m22 * cxz
    n12 = c12x * c12x + c12y * c12y + c12z * c12z
    n13 = c13x * c13x + c13y * c13y + c13z * c13z
    n23 = c23x * c23x + c23y * c23y + c23z * c23z
    use12 = jnp.logical_and(n12 >= n13, n12 >= n23)
    use13 = jnp.logical_and(jnp.logical_not(use12), n13 >= n23)
    vx = jnp.where(use12, c12x, jnp.where(use13, c13x, c23x))
    vy = jnp.where(use12, c12y, jnp.where(use13, c13y, c23y))
    vz = jnp.where(use12, c12z, jnp.where(use13, c13z, c23z))
    inv = jax.lax.rsqrt(vx * vx + vy * vy + vz * vz + 1e-38)
    return vx * inv, vy * inv, vz * inv


def _body(prow_p_ref, pall_p_ref, pallT_p_ref,
          prow_g_ref, pall_g_ref, pallT_g_ref,
          out_ref, fphi, fplo, fghi, fglo, acc):
    rb = pl.program_id(1)

    @pl.when(rb == 0)
    def _():
        _build_feats(pall_p_ref[0], fphi, fplo)
        _build_feats(pall_g_ref[0], fghi, fglo)
        acc[0, 0] = 0.0

    pnx, pny, pnz = _normals(prow_p_ref[0], pallT_p_ref[0], fphi, fplo)
    gnx, gny, gnz = _normals(prow_g_ref[0], pallT_g_ref[0], fghi, fglo)
    cos = pnx * gnx + pny * gny + pnz * gnz
    acc[0, 0] += jnp.sum(1.0 - jnp.abs(cos))

    @pl.when(rb == _NB - 1)
    def _():
        out_ref[...] = jnp.full((1, 1, _L), acc[0, 0], jnp.float32)


def kernel(pred, gt):
    pts = jnp.concatenate([pred, gt], axis=0)
    p = jnp.transpose(pts, (0, 2, 1))
    ppad = jnp.pad(p, ((0, 0), (0, 0), (0, _L - 3)))
    ppadT = jnp.transpose(ppad, (0, 2, 1))

    partials = pl.pallas_call(
        _body,
        grid=(4, _NB),
        in_specs=[
            pl.BlockSpec((1, _R, _L), lambda c, rb: (c, rb, 0)),
            pl.BlockSpec((1, _N, _L), lambda c, rb: (c, 0, 0)),
            pl.BlockSpec((1, _L, _N), lambda c, rb: (c, 0, 0)),
            pl.BlockSpec((1, _R, _L), lambda c, rb: (c + 4, rb, 0)),
            pl.BlockSpec((1, _N, _L), lambda c, rb: (c + 4, 0, 0)),
            pl.BlockSpec((1, _L, _N), lambda c, rb: (c + 4, 0, 0)),
        ],
        out_specs=pl.BlockSpec((1, 1, _L), lambda c, rb: (c, 0, 0)),
        out_shape=jax.ShapeDtypeStruct((4, 1, _L), jnp.float32),
        scratch_shapes=[
            pltpu.VMEM((_N, _L), jnp.bfloat16),
            pltpu.VMEM((_N, _L), jnp.bfloat16),
            pltpu.VMEM((_N, _L), jnp.bfloat16),
            pltpu.VMEM((_N, _L), jnp.bfloat16),
            pltpu.SMEM((1, 1), jnp.float32),
        ],
        compiler_params=pltpu.CompilerParams(
            dimension_semantics=("parallel", "arbitrary")),
        interpret=_INTERPRET,
    )(ppad, ppad, ppadT, ppad, ppad, ppadT)
    return jnp.sum(partials[:, 0, 0]) * (1.0 / (4.0 * _N))


# speedup vs baseline: 1.8489x; 1.8489x over previous
import jax
import jax.numpy as jnp
from jax.experimental import pallas as pl
from jax.experimental.pallas import tpu as pltpu

_K = 10
_N = 2048
_R = 256
_NB = _N // _R
_L = 128
_INTERPRET = False


def _build_feats(pall, fhi_ref, flo_ref):
    x = pall[:, 0:1]
    y = pall[:, 1:2]
    z = pall[:, 2:3]
    f = jnp.concatenate(
        [x, y, z, x * x, y * y, z * z, x * y, x * z, y * z,
         jnp.ones((_N, 1), jnp.float32),
         jnp.zeros((_N, _L - 10), jnp.float32)], axis=1)
    hi = f.astype(jnp.bfloat16)
    fhi_ref[...] = hi
    flo_ref[...] = (f - hi.astype(jnp.float32)).astype(jnp.bfloat16)


def _normals(prow, pallT, fhi_ref, flo_ref):
    g = jax.lax.dot_general(prow, pallT, (((1,), (0,)), ((), ())),
                            preferred_element_type=jnp.float32)
    sq_row = jnp.sum(prow * prow, axis=1, keepdims=True)
    sq_all = jnp.sum(pallT * pallT, axis=0, keepdims=True)
    d = sq_row - 2.0 * g + sq_all

    lo = jnp.minimum(d[:, 0:128], d[:, 128:256])
    hi = jnp.maximum(d[:, 0:128], d[:, 128:256])
    for j in range(2, _N // 128):
        v = d[:, 128 * j:128 * (j + 1)]
        hi = jnp.minimum(hi, jnp.maximum(lo, v))
        lo = jnp.minimum(lo, v)
    w = jnp.concatenate([lo, hi], axis=1)
    t = jnp.min(w, axis=1, keepdims=True)
    for _ in range(_K - 1):
        t = jnp.min(jnp.where(w > t, w, jnp.inf), axis=1, keepdims=True)
    mask = (d <= t).astype(jnp.bfloat16)

    dims = (((1,), (0,)), ((), ()))
    s = (jax.lax.dot_general(mask, fhi_ref[...], dims,
                             preferred_element_type=jnp.float32)
         + jax.lax.dot_general(mask, flo_ref[...], dims,
                               preferred_element_type=jnp.float32))

    st = jnp.transpose(s)
    inv_c = 1.0 / st[9:10, :]
    mx = st[0:1, :] * inv_c
    my = st[1:2, :] * inv_c
    mz = st[2:3, :] * inv_c
    cxx = st[3:4, :] * inv_c - mx * mx
    cyy = st[4:5, :] * inv_c - my * my
    czz = st[5:6, :] * inv_c - mz * mz
    cxy = st[6:7, :] * inv_c - mx * my
    cxz = st[7:8, :] * inv_c - mx * mz
    cyz = st[8:9, :] * inv_c - my * mz

    q = (cxx + cyy + czz) * (1.0 / 3.0)
    aa = cxx - q
    bb = cyy - q
    cc = czz - q
    p2 = aa * aa + bb * bb + cc * cc + 2.0 * (cxy * cxy + cxz * cxz + cyz * cyz)
    p = jnp.sqrt(p2 * (1.0 / 6.0) + 1e-38)
    pinv = 1.0 / p
    b11 = aa * pinv
    b22 = bb * pinv
    b33 = cc * pinv
    b12 = cxy * pinv
    b13 = cxz * pinv
    b23 = cyz * pinv
    detb = (b11 * (b22 * b33 - b23 * b23)
            - b12 * (b12 * b33 - b23 * b13)
            + b13 * (b12 * b23 - b22 * b13))
    r = jnp.clip(detb * 0.5, -1.0, 1.0)
    lam = jnp.full_like(r, -2.0)
    for _ in range(12):
        f = lam * lam * lam - 3.0 * lam - 2.0 * r
        fp = 3.0 * lam * lam - 3.0 + 1e-10
        lam = lam - f / fp
    lmin = q + p * lam

    m11 = cxx - lmin
    m22 = cyy - lmin
    m33 = czz - lmin
    c12x = cxy * cyz - cxz * m22
    c12y = cxz * cxy - m11 * cyz
    c12z = m11 * m22 - cxy * cxy
    c13x = cxy * m33 - cxz * cyz
    c13y = cxz * cxz - m11 * m33
    c13z = m11 * cyz - cxy * cxz
    c23x = m22 * m33 - cyz * cyz
    c23y = cyz * cxz - cxy * m33
    c23z = cxy * cyz - m22 * cxz
    n12 = c12x * c12x + c12y * c12y + c12z * c12z
    n13 = c13x * c13x + c13y * c13y + c13z * c13z
    n23 = c23x * c23x + c23y * c23y + c23z * c23z
    use12 = jnp.logical_and(n12 >= n13, n12 >= n23)
    use13 = jnp.logical_and(jnp.logical_not(use12), n13 >= n23)
    vx = jnp.where(use12, c12x, jnp.where(use13, c13x, c23x))
    vy = jnp.where(use12, c12y, jnp.where(use13, c13y, c23y))
    vz = jnp.where(use12, c12z, jnp.where(use13, c13z, c23z))
    inv = jax.lax.rsqrt(vx * vx + vy * vy + vz * vz + 1e-38)
    return vx * inv, vy * inv, vz * inv


def _body(prow_p_ref, pall_p_ref, pallT_p_ref,
          prow_g_ref, pall_g_ref, pallT_g_ref,
          out_ref, fphi, fplo, fghi, fglo, acc):
    rb = pl.program_id(1)

    @pl.when(rb == 0)
    def _():
        _build_feats(pall_p_ref[0], fphi, fplo)
        _build_feats(pall_g_ref[0], fghi, fglo)
        acc[0, 0] = 0.0

    pnx, pny, pnz = _normals(prow_p_ref[0], pallT_p_ref[0], fphi, fplo)
    gnx, gny, gnz = _normals(prow_g_ref[0], pallT_g_ref[0], fghi, fglo)
    cos = pnx * gnx + pny * gny + pnz * gnz
    acc[0, 0] += jnp.sum(1.0 - jnp.abs(cos))

    @pl.when(rb == _NB - 1)
    def _():
        out_ref[...] = jnp.full((1, 1, _L), acc[0, 0], jnp.float32)


def kernel(pred, gt):
    pts = jnp.concatenate([pred, gt], axis=0)
    p = jnp.transpose(pts, (0, 2, 1))
    ppad = jnp.pad(p, ((0, 0), (0, 0), (0, _L - 3)))
    ppadT = jnp.transpose(ppad, (0, 2, 1))

    partials = pl.pallas_call(
        _body,
        grid=(4, _NB),
        in_specs=[
            pl.BlockSpec((1, _R, _L), lambda c, rb: (c, rb, 0)),
            pl.BlockSpec((1, _N, _L), lambda c, rb: (c, 0, 0)),
            pl.BlockSpec((1, _L, _N), lambda c, rb: (c, 0, 0)),
            pl.BlockSpec((1, _R, _L), lambda c, rb: (c + 4, rb, 0)),
            pl.BlockSpec((1, _N, _L), lambda c, rb: (c + 4, 0, 0)),
            pl.BlockSpec((1, _L, _N), lambda c, rb: (c + 4, 0, 0)),
        ],
        out_specs=pl.BlockSpec((1, 1, _L), lambda c, rb: (c, 0, 0)),
        out_shape=jax.ShapeDtypeStruct((4, 1, _L), jnp.float32),
        scratch_shapes=[
            pltpu.VMEM((_N, _L), jnp.bfloat16),
            pltpu.VMEM((_N, _L), jnp.bfloat16),
            pltpu.VMEM((_N, _L), jnp.bfloat16),
            pltpu.VMEM((_N, _L), jnp.bfloat16),
            pltpu.SMEM((1, 1), jnp.float32),
        ],
        compiler_params=pltpu.CompilerParams(
            dimension_semantics=("parallel", "arbitrary")),
        interpret=_INTERPRET,
    )(ppad, ppad, ppadT, ppad, ppad, ppadT)
    return jnp.sum(partials[:, 0, 0]) * (1.0 / (4.0 * _N))


# speedup vs baseline: 2.4354x; 1.3172x over previous
import jax
import jax.numpy as jnp
from jax.experimental import pallas as pl
from jax.experimental.pallas import tpu as pltpu

_K = 10
_N = 2048
_R = 512
_NB = _N // _R
_L = 128
_INTERPRET = False


def _build_feats(pall, fhi_ref, flo_ref):
    x = pall[:, 0:1]
    y = pall[:, 1:2]
    z = pall[:, 2:3]
    f = jnp.concatenate(
        [x, y, z, x * x, y * y, z * z, x * y, x * z, y * z,
         jnp.ones((_N, 1), jnp.float32),
         jnp.zeros((_N, _L - 10), jnp.float32)], axis=1)
    hi = f.astype(jnp.bfloat16)
    fhi_ref[...] = hi
    flo_ref[...] = (f - hi.astype(jnp.float32)).astype(jnp.bfloat16)


def _normals(prow, pallT, fhi_ref, flo_ref, sqa_ref):
    g = jax.lax.dot_general(prow, pallT, (((1,), (0,)), ((), ())),
                            preferred_element_type=jnp.float32)
    d = sqa_ref[...] - 2.0 * g

    los = []
    his = []
    for j in range(0, _N // 128, 2):
        a = d[:, 128 * j:128 * (j + 1)]
        b = d[:, 128 * (j + 1):128 * (j + 2)]
        los.append(jnp.minimum(a, b))
        his.append(jnp.maximum(a, b))
    while len(los) > 1:
        nlo = []
        nhi = []
        for i in range(0, len(los), 2):
            l1, h1 = los[i], his[i]
            l2, h2 = los[i + 1], his[i + 1]
            nlo.append(jnp.minimum(l1, l2))
            nhi.append(jnp.minimum(jnp.maximum(l1, l2), jnp.minimum(h1, h2)))
        los, his = nlo, nhi
    w = jnp.concatenate([los[0], his[0]], axis=1)
    t = jnp.min(w, axis=1, keepdims=True)
    for _ in range(_K - 1):
        t = jnp.min(jnp.where(w > t, w, jnp.inf), axis=1, keepdims=True)
    mask = (d <= t).astype(jnp.bfloat16)

    dims = (((1,), (0,)), ((), ()))
    s = (jax.lax.dot_general(mask, fhi_ref[...], dims,
                             preferred_element_type=jnp.float32)
         + jax.lax.dot_general(mask, flo_ref[...], dims,
                               preferred_element_type=jnp.float32))

    st = jnp.transpose(s)
    inv_c = 1.0 / st[9:10, :]
    mx = st[0:1, :] * inv_c
    my = st[1:2, :] * inv_c
    mz = st[2:3, :] * inv_c
    cxx = st[3:4, :] * inv_c - mx * mx
    cyy = st[4:5, :] * inv_c - my * my
    czz = st[5:6, :] * inv_c - mz * mz
    cxy = st[6:7, :] * inv_c - mx * my
    cxz = st[7:8, :] * inv_c - mx * mz
    cyz = st[8:9, :] * inv_c - my * mz

    q = (cxx + cyy + czz) * (1.0 / 3.0)
    aa = cxx - q
    bb = cyy - q
    cc = czz - q
    p2 = aa * aa + bb * bb + cc * cc + 2.0 * (cxy * cxy + cxz * cxz + cyz * cyz)
    p = jnp.sqrt(p2 * (1.0 / 6.0) + 1e-38)
    pinv = 1.0 / p
    b11 = aa * pinv
    b22 = bb * pinv
    b33 = cc * pinv
    b12 = cxy * pinv
    b13 = cxz * pinv
    b23 = cyz * pinv
    detb = (b11 * (b22 * b33 - b23 * b23)
            - b12 * (b12 * b33 - b23 * b13)
            + b13 * (b12 * b23 - b22 * b13))
    r = jnp.clip(detb * 0.5, -1.0, 1.0)
    lam = jnp.full_like(r, -2.0)
    for _ in range(12):
        f = lam * lam * lam - 3.0 * lam - 2.0 * r
        fp = 3.0 * lam * lam - 3.0 + 1e-10
        lam = lam - f / fp
    lmin = q + p * lam

    m11 = cxx - lmin
    m22 = cyy - lmin
    m33 = czz - lmin
    c12x = cxy * cyz - cxz * m22
    c12y = cxz * cxy - m11 * cyz
    c12z = m11 * m22 - cxy * cxy
    c13x = cxy * m33 - cxz * cyz
    c13y = cxz * cxz - m11 * m33
    c13z = m11 * cyz - cxy * cxz
    c23x = m22 * m33 - cyz * cyz
    c23y = cyz * cxz - cxy * m33
    c23z = cxy * cyz - m22 * cxz
    n12 = c12x * c12x + c12y * c12y + c12z * c12z
    n13 = c13x * c13x + c13y * c13y + c13z * c13z
    n23 = c23x * c23x + c23y * c23y + c23z * c23z
    use12 = jnp.logical_and(n12 >= n13, n12 >= n23)
    use13 = jnp.logical_and(jnp.logical_not(use12), n13 >= n23)
    vx = jnp.where(use12, c12x, jnp.where(use13, c13x, c23x))
    vy = jnp.where(use12, c12y, jnp.where(use13, c13y, c23y))
    vz = jnp.where(use12, c12z, jnp.where(use13, c13z, c23z))
    inv = jax.lax.rsqrt(vx * vx + vy * vy + vz * vz + 1e-38)
    return vx * inv, vy * inv, vz * inv


def _body(prow_p_ref, pall_p_ref, pallT_p_ref,
          prow_g_ref, pall_g_ref, pallT_g_ref,
          out_ref, fphi, fplo, fghi, fglo, sqap, sqag, acc):
    rb = pl.program_id(1)

    @pl.when(rb == 0)
    def _():
        _build_feats(pall_p_ref[0], fphi, fplo)
        _build_feats(pall_g_ref[0], fghi, fglo)
        tp = pallT_p_ref[0]
        tg = pallT_g_ref[0]
        sqap[...] = jnp.sum(tp * tp, axis=0, keepdims=True)
        sqag[...] = jnp.sum(tg * tg, axis=0, keepdims=True)
        acc[0, 0] = 0.0

    pnx, pny, pnz = _normals(prow_p_ref[0], pallT_p_ref[0], fphi, fplo, sqap)
    gnx, gny, gnz = _normals(prow_g_ref[0], pallT_g_ref[0], fghi, fglo, sqag)
    cos = pnx * gnx + pny * gny + pnz * gnz
    acc[0, 0] += jnp.sum(1.0 - jnp.abs(cos))

    @pl.when(rb == _NB - 1)
    def _():
        out_ref[...] = jnp.full((1, 1, _L), acc[0, 0], jnp.float32)


def kernel(pred, gt):
    pts = jnp.concatenate([pred, gt], axis=0)
    p = jnp.transpose(pts, (0, 2, 1))
    ppad = jnp.pad(p, ((0, 0), (0, 0), (0, _L - 3)))
    ppadT = jnp.transpose(ppad, (0, 2, 1))

    partials = pl.pallas_call(
        _body,
        grid=(4, _NB),
        in_specs=[
            pl.BlockSpec((1, _R, _L), lambda c, rb: (c, rb, 0)),
            pl.BlockSpec((1, _N, _L), lambda c, rb: (c, 0, 0)),
            pl.BlockSpec((1, _L, _N), lambda c, rb: (c, 0, 0)),
            pl.BlockSpec((1, _R, _L), lambda c, rb: (c + 4, rb, 0)),
            pl.BlockSpec((1, _N, _L), lambda c, rb: (c + 4, 0, 0)),
            pl.BlockSpec((1, _L, _N), lambda c, rb: (c + 4, 0, 0)),
        ],
        out_specs=pl.BlockSpec((1, 1, _L), lambda c, rb: (c, 0, 0)),
        out_shape=jax.ShapeDtypeStruct((4, 1, _L), jnp.float32),
        scratch_shapes=[
            pltpu.VMEM((_N, _L), jnp.bfloat16),
            pltpu.VMEM((_N, _L), jnp.bfloat16),
            pltpu.VMEM((_N, _L), jnp.bfloat16),
            pltpu.VMEM((_N, _L), jnp.bfloat16),
            pltpu.VMEM((1, _N), jnp.float32),
            pltpu.VMEM((1, _N), jnp.float32),
            pltpu.SMEM((1, 1), jnp.float32),
        ],
        compiler_params=pltpu.CompilerParams(
            dimension_semantics=("parallel", "arbitrary")),
        interpret=_INTERPRET,
    )(ppad, ppad, ppadT, ppad, ppad, ppadT)
    return jnp.sum(partials[:, 0, 0]) * (1.0 / (4.0 * _N))


# speedup vs baseline: 2.6834x; 1.1019x over previous
import jax
import jax.numpy as jnp
from jax.experimental import pallas as pl
from jax.experimental.pallas import tpu as pltpu

_K = 10
_N = 2048
_R = 512
_NB = _N // _R
_L = 128
_C = 16
_INTERPRET = False


def _build_feats(pallT, fhi_ref, flo_ref):
    x = pallT[0:1, :]
    y = pallT[1:2, :]
    z = pallT[2:3, :]
    fT = jnp.concatenate(
        [x, y, z, x * x, y * y, z * z, x * y, x * z, y * z,
         jnp.ones((1, _N), jnp.float32),
         jnp.zeros((_C - 10, _N), jnp.float32)], axis=0)
    hiT = fT.astype(jnp.bfloat16)
    loT = (fT - hiT.astype(jnp.float32)).astype(jnp.bfloat16)
    fhi_ref[...] = jnp.transpose(hiT)
    flo_ref[...] = jnp.transpose(loT)


def _normals(prow, pallT, fhi_ref, flo_ref, sqa_ref):
    g = jax.lax.dot_general(prow, pallT, (((1,), (0,)), ((), ())),
                            preferred_element_type=jnp.float32)

    def dch(j):
        sl = slice(128 * j, 128 * (j + 1))
        return sqa_ref[:, sl] - g[:, sl]

    los = []
    his = []
    for j in range(0, _N // 128, 2):
        a = dch(j)
        b = dch(j + 1)
        los.append(jnp.minimum(a, b))
        his.append(jnp.maximum(a, b))
    while len(los) > 1:
        nlo = []
        nhi = []
        for i in range(0, len(los), 2):
            l1, h1 = los[i], his[i]
            l2, h2 = los[i + 1], his[i + 1]
            nlo.append(jnp.minimum(l1, l2))
            nhi.append(jnp.minimum(jnp.maximum(l1, l2), jnp.minimum(h1, h2)))
        los, his = nlo, nhi
    w = jnp.concatenate([los[0], his[0]], axis=1)
    t = jnp.min(w, axis=1, keepdims=True)
    for _ in range(_K - 1):
        t = jnp.min(jnp.where(w > t, w, jnp.inf), axis=1, keepdims=True)
    mask = jnp.concatenate(
        [(dch(j) <= t) for j in range(_N // 128)],
        axis=1).astype(jnp.bfloat16)

    dims = (((1,), (0,)), ((), ()))
    s = (jax.lax.dot_general(mask, fhi_ref[...], dims,
                             preferred_element_type=jnp.float32)
         + jax.lax.dot_general(mask, flo_ref[...], dims,
                               preferred_element_type=jnp.float32))

    st = jnp.transpose(s)
    inv_c = 1.0 / st[9:10, :]
    mx = st[0:1, :] * inv_c
    my = st[1:2, :] * inv_c
    mz = st[2:3, :] * inv_c
    cxx = st[3:4, :] * inv_c - mx * mx
    cyy = st[4:5, :] * inv_c - my * my
    czz = st[5:6, :] * inv_c - mz * mz
    cxy = st[6:7, :] * inv_c - mx * my
    cxz = st[7:8, :] * inv_c - mx * mz
    cyz = st[8:9, :] * inv_c - my * mz

    q = (cxx + cyy + czz) * (1.0 / 3.0)
    aa = cxx - q
    bb = cyy - q
    cc = czz - q
    p2 = aa * aa + bb * bb + cc * cc + 2.0 * (cxy * cxy + cxz * cxz + cyz * cyz)
    p = jnp.sqrt(p2 * (1.0 / 6.0) + 1e-38)
    pinv = 1.0 / p
    b11 = aa * pinv
    b22 = bb * pinv
    b33 = cc * pinv
    b12 = cxy * pinv
    b13 = cxz * pinv
    b23 = cyz * pinv
    detb = (b11 * (b22 * b33 - b23 * b23)
            - b12 * (b12 * b33 - b23 * b13)
            + b13 * (b12 * b23 - b22 * b13))
    r = jnp.clip(detb * 0.5, -1.0, 1.0)
    lam = jnp.full_like(r, -2.0)
    for _ in range(12):
        f = lam * lam * lam - 3.0 * lam - 2.0 * r
        fp = 3.0 * lam * lam - 3.0 + 1e-10
        lam = lam - f / fp
    lmin = q + p * lam

    m11 = cxx - lmin
    m22 = cyy - lmin
    m33 = czz - lmin
    c12x = cxy * cyz - cxz * m22
    c12y = cxz * cxy - m11 * cyz
    c12z = m11 * m22 - cxy * cxy
    c13x = cxy * m33 - cxz * cyz
    c13y = cxz * cxz - m11 * m33
    c13z = m11 * cyz - cxy * cxz
    c23x = m22 * m33 - cyz * cyz
    c23y = cyz * cxz - cxy * m33
    c23z = cxy * cyz - m22 * cxz
    n12 = c12x * c12x + c12y * c12y + c12z * c12z
    n13 = c13x * c13x + c13y * c13y + c13z * c13z
    n23 = c23x * c23x + c23y * c23y + c23z * c23z
    use12 = jnp.logical_and(n12 >= n13, n12 >= n23)
    use13 = jnp.logical_and(jnp.logical_not(use12), n13 >= n23)
    vx = jnp.where(use12, c12x, jnp.where(use13, c13x, c23x))
    vy = jnp.where(use12, c12y, jnp.where(use13, c13y, c23y))
    vz = jnp.where(use12, c12z, jnp.where(use13, c13z, c23z))
    inv = jax.lax.rsqrt(vx * vx + vy * vy + vz * vz + 1e-38)
    return vx * inv, vy * inv, vz * inv


def _body(prow_p_ref, pallT_p_ref,
          prow_g_ref, pallT_g_ref,
          out_ref, fphi, fplo, fghi, fglo, sqap, sqag, acc):
    rb = pl.program_id(1)

    @pl.when(rb == 0)
    def _():
        tp = pallT_p_ref[0]
        tg = pallT_g_ref[0]
        _build_feats(tp, fphi, fplo)
        _build_feats(tg, fghi, fglo)
        sqap[...] = 0.5 * jnp.sum(tp * tp, axis=0, keepdims=True)
        sqag[...] = 0.5 * jnp.sum(tg * tg, axis=0, keepdims=True)
        acc[0, 0] = 0.0

    pnx, pny, pnz = _normals(prow_p_ref[0], pallT_p_ref[0], fphi, fplo, sqap)
    gnx, gny, gnz = _normals(prow_g_ref[0], pallT_g_ref[0], fghi, fglo, sqag)
    cos = pnx * gnx + pny * gny + pnz * gnz
    acc[0, 0] += jnp.sum(1.0 - jnp.abs(cos))

    @pl.when(rb == _NB - 1)
    def _():
        out_ref[...] = jnp.full((1, 1, _L), acc[0, 0], jnp.float32)


def kernel(pred, gt):
    pts = jnp.concatenate([pred, gt], axis=0)
    p = jnp.transpose(pts, (0, 2, 1))
    ppad = jnp.pad(p, ((0, 0), (0, 0), (0, _L - 3)))
    ppadT = jnp.transpose(ppad, (0, 2, 1))

    partials = pl.pallas_call(
        _body,
        grid=(4, _NB),
        in_specs=[
            pl.BlockSpec((1, _R, _L), lambda c, rb: (c, rb, 0)),
            pl.BlockSpec((1, _L, _N), lambda c, rb: (c, 0, 0)),
            pl.BlockSpec((1, _R, _L), lambda c, rb: (c + 4, rb, 0)),
            pl.BlockSpec((1, _L, _N), lambda c, rb: (c + 4, 0, 0)),
        ],
        out_specs=pl.BlockSpec((1, 1, _L), lambda c, rb: (c, 0, 0)),
        out_shape=jax.ShapeDtypeStruct((4, 1, _L), jnp.float32),
        scratch_shapes=[
            pltpu.VMEM((_N, _C), jnp.bfloat16),
            pltpu.VMEM((_N, _C), jnp.bfloat16),
            pltpu.VMEM((_N, _C), jnp.bfloat16),
            pltpu.VMEM((_N, _C), jnp.bfloat16),
            pltpu.VMEM((1, _N), jnp.float32),
            pltpu.VMEM((1, _N), jnp.float32),
            pltpu.SMEM((1, 1), jnp.float32),
        ],
        compiler_params=pltpu.CompilerParams(
            dimension_semantics=("parallel", "arbitrary")),
        interpret=_INTERPRET,
    )(ppad, ppadT, ppad, ppadT)
    return jnp.sum(partials[:, 0, 0]) * (1.0 / (4.0 * _N))
